# baseline (speedup 1.0000x reference)
import jax
import jax.numpy as jnp
from jax import lax
from jax.experimental import pallas as pl

_N = 10000
_NGRAPH = 100
_NPG = 100
_K = 4
_BG = 2

_f32 = jnp.float32
_bf16 = jnp.bfloat16
_EXACT = lax.Precision.HIGHEST


def _lrelu(h):
    return jnp.where(h >= 0, h, 0.01 * h)


def _bdot(a, w_bf16):
    return jnp.dot(a.astype(_bf16), w_bf16, preferred_element_type=_f32)


def _topk_sels(pos):
    n = _NPG
    pos_t = jnp.transpose(pos)
    d2 = None
    for c in range(3):
        dc = pos[:, c:c + 1] - pos_t[c:c + 1, :]
        dc = dc * dc
        d2 = dc if d2 is None else d2 + dc
    ii = lax.broadcasted_iota(jnp.int32, (n, n), 0)
    jj = lax.broadcasted_iota(jnp.int32, (n, n), 1)
    d2 = d2 + jnp.where(ii == jj, _f32(1e10), _f32(0.0))
    idxs = []
    for _ in range(_K):
        m = jnp.min(d2, axis=1, keepdims=True)
        cand = jnp.where(d2 <= m, jj, n)
        idx = jnp.min(cand, axis=1, keepdims=True)
        idxs.append(idx)
        d2 = jnp.where(jj == idx, _f32(jnp.inf), d2)
    idx_cat = jnp.concatenate(idxs, axis=0)
    jj4 = lax.broadcasted_iota(jnp.int32, (_K * n, n), 1)
    return (jj4 == idx_cat).astype(_bf16)


def _edge_conv(feat, sel4, w1, b1, w2, b2):
    n = _NPG
    hi = feat.astype(_bf16)
    r1 = feat - hi.astype(_f32)
    md = r1.astype(_bf16)
    lo = (r1 - md.astype(_f32)).astype(_bf16)
    xj = (jnp.dot(sel4, hi, preferred_element_type=_f32)
          + jnp.dot(sel4, md, preferred_element_type=_f32)
          + jnp.dot(sel4, lo, preferred_element_type=_f32))
    xi = jnp.concatenate([feat] * _K, axis=0)
    e = jnp.concatenate([xi, xj - xi], axis=1)
    h = _lrelu(_bdot(e, w1) + b1)
    h = _lrelu(_bdot(h, w2) + b2)
    return ((h[0 * n:1 * n] + h[1 * n:2 * n])
            + h[2 * n:3 * n]) + h[3 * n:4 * n]


def _body(x_ref,
          c1w1, c1b1, c1w2, c1b2,
          c2w1, c2b1, c2w2, c2b2,
          v1w1, v1b1, v1w2, v1b2,
          v2w1, v2b1, v2w2, v2b2,
          v3w1, v3b1, v3w2, v3b2,
          v4w1, v4b1, v4w2, v4b2,
          n1wx, n1wa, n1wb, n1wc, n1wd, n1b,
          n2w, n2b,
          n3wa, n3wb, n3wc, n3wd, n3b,
          n4w, n4b,
          o_ref):
    for g in range(_BG):
        o_ref[g] = _graph(
            x_ref[g],
            c1w1, c1b1, c1w2, c1b2, c2w1, c2b1, c2w2, c2b2,
            v1w1, v1b1, v1w2, v1b2, v2w1, v2b1, v2w2, v2b2,
            v3w1, v3b1, v3w2, v3b2, v4w1, v4b1, v4w2, v4b2,
            n1wx, n1wa, n1wb, n1wc, n1wd, n1b, n2w, n2b,
            n3wa, n3wb, n3wc, n3wd, n3b, n4w, n4b)


def _graph(x,
           c1w1, c1b1, c1w2, c1b2,
           c2w1, c2b1, c2w2, c2b2,
           v1w1, v1b1, v1w2, v1b2,
           v2w1, v2b1, v2w2, v2b2,
           v3w1, v3b1, v3w2, v3b2,
           v4w1, v4b1, v4w2, v4b2,
           n1wx, n1wa, n1wb, n1wc, n1wd, n1b,
           n2w, n2b,
           n3wa, n3wb, n3wc, n3wd, n3b,
           n4w, n4b):
    h = _lrelu(_bdot(x, c1w1[...]) + c1b1[...])
    h = _lrelu(_bdot(h, c1w2[...]) + c1b2[...])
    h = _lrelu(h)
    h = _lrelu(_bdot(h, c2w1[...]) + c2b1[...])
    h = _lrelu(_bdot(h, c2w2[...]) + c2b2[...])
    xm = x * (h > 0).astype(_f32)

    sels = _topk_sels(xm[:, 0:3])
    a = _edge_conv(xm, sels, v1w1[...], v1b1[...], v1w2[...], v1b2[...])
    sels = _topk_sels(a[:, 0:3])
    b = _edge_conv(a, sels, v2w1[...], v2b1[...], v2w2[...], v2b2[...])
    sels = _topk_sels(b[:, 0:3])
    c = _edge_conv(b, sels, v3w1[...], v3b1[...], v3w2[...], v3b2[...])
    sels = _topk_sels(c[:, 0:3])
    d = _edge_conv(c, sels, v4w1[...], v4b1[...], v4w2[...], v4b2[...])

    h = (_bdot(xm, n1wx[...]) + _bdot(a, n1wa[...]) + _bdot(b, n1wb[...])
         + _bdot(c, n1wc[...]) + _bdot(d, n1wd[...]) + n1b[...])
    h = _lrelu(h)
    h = _bdot(h, n2w[...]) + n2b[...]

    ga = jnp.max(h, axis=0, keepdims=True)
    gb = jnp.min(h, axis=0, keepdims=True)
    gc = jnp.sum(h, axis=0, keepdims=True)
    gd = gc / _f32(_NPG)
    t = (_bdot(_lrelu(ga), n3wa[...]) + _bdot(_lrelu(gb), n3wb[...])
         + _bdot(_lrelu(gc), n3wc[...]) + _bdot(_lrelu(gd), n3wd[...])
         + n3b[...])
    t = _lrelu(t)
    out = _bdot(t, n4w[...]) + n4b[...]
    lane = lax.broadcasted_iota(jnp.int32, (1, 3), 1)
    return jnp.where(lane < 2, jnp.tanh(out), out)


def kernel(x, edge_index, batch, params):
    del edge_index, batch
    p = params

    def full(arr):
        return pl.BlockSpec(arr.shape, lambda *_: (0,) * arr.ndim)

    def b2(name):
        return p[name].reshape(1, -1)

    def wb(arr):
        return arr.astype(_bf16)

    weights = []
    for pre in ('clean1', 'clean2'):
        weights += [wb(p[pre + '_W1']), b2(pre + '_b1'),
                    wb(p[pre + '_W2']), b2(pre + '_b2')]
    for pre in ('conv1', 'conv2', 'conv3', 'conv4'):
        weights += [wb(p[pre + '_W1']), b2(pre + '_b1'),
                    wb(p[pre + '_W2']), b2(pre + '_b2')]
    w = p['nn1_W']
    weights += [wb(w[0:5]), wb(w[5:197]), wb(w[197:389]), wb(w[389:581]),
                wb(w[581:773]), b2('nn1_b')]
    weights += [wb(p['nn2_W']), b2('nn2_b')]
    w = p['nn3_W']
    weights += [wb(w[0:192]), wb(w[192:384]), wb(w[384:576]), wb(w[576:768]),
                b2('nn3_b')]
    weights += [wb(p['nn4_W']), b2('nn4_b')]

    x3 = x.reshape(_NGRAPH, _NPG, 5)
    out = pl.pallas_call(
        _body,
        grid=(_NGRAPH // _BG,),
        in_specs=[pl.BlockSpec((_BG, _NPG, 5), lambda i: (i, 0, 0))]
                 + [full(wt) for wt in weights],
        out_specs=pl.BlockSpec((_BG, 1, 3), lambda i: (i, 0, 0)),
        out_shape=jax.ShapeDtypeStruct((_NGRAPH, 1, 3), _f32),
    )(x3, *weights)
    return out.reshape(_NGRAPH, 3)


# speedup vs baseline: 1.7229x; 1.7229x over previous
import jax
import jax.numpy as jnp
from jax import lax
from jax.experimental import pallas as pl

_N = 10000
_NGRAPH = 100
_NPG = 100
_K = 4
_BG = 4

_f32 = jnp.float32
_bf16 = jnp.bfloat16
_EXACT = lax.Precision.HIGHEST


def _lrelu(h):
    return jnp.where(h >= 0, h, 0.01 * h)


def _bdot(a, w_bf16):
    return jnp.dot(a.astype(_bf16), w_bf16, preferred_element_type=_f32)


def _topk_sels(pos):
    n = _NPG
    pos_t = jnp.transpose(pos)
    d2 = None
    for c in range(3):
        dc = pos[:, c:c + 1] - pos_t[c:c + 1, :]
        dc = dc * dc
        d2 = dc if d2 is None else d2 + dc
    ii = lax.broadcasted_iota(jnp.int32, (n, n), 0)
    jj = lax.broadcasted_iota(jnp.int32, (n, n), 1)
    d2 = d2 + jnp.where(ii == jj, _f32(1e10), _f32(0.0))
    idxs = []
    for _ in range(_K):
        m = jnp.min(d2, axis=1, keepdims=True)
        cand = jnp.where(d2 <= m, jj, n)
        idx = jnp.min(cand, axis=1, keepdims=True)
        idxs.append(idx)
        d2 = jnp.where(jj == idx, _f32(jnp.inf), d2)
    idx_cat = jnp.concatenate(idxs, axis=0)
    jj4 = lax.broadcasted_iota(jnp.int32, (_K * n, n), 1)
    return (jj4 == idx_cat).astype(_bf16)


def _edge_conv(feat, sel4, w1, b1, w2, b2):
    n = _NPG
    hi = feat.astype(_bf16)
    r1 = feat - hi.astype(_f32)
    md = r1.astype(_bf16)
    lo = (r1 - md.astype(_f32)).astype(_bf16)
    xj = (jnp.dot(sel4, hi, preferred_element_type=_f32)
          + jnp.dot(sel4, md, preferred_element_type=_f32)
          + jnp.dot(sel4, lo, preferred_element_type=_f32))
    xi = jnp.concatenate([feat] * _K, axis=0)
    e = jnp.concatenate([xi, xj - xi], axis=1)
    h = _lrelu(_bdot(e, w1) + b1)
    h = _lrelu(_bdot(h, w2) + b2)
    return ((h[0 * n:1 * n] + h[1 * n:2 * n])
            + h[2 * n:3 * n]) + h[3 * n:4 * n]


def _body(x_ref,
          c1w1, c1b1, c1w2, c1b2,
          c2w1, c2b1, c2w2, c2b2,
          v1w1, v1b1, v1w2, v1b2,
          v2w1, v2b1, v2w2, v2b2,
          v3w1, v3b1, v3w2, v3b2,
          v4w1, v4b1, v4w2, v4b2,
          n1wx, n1wa, n1wb, n1wc, n1wd, n1b,
          n2w, n2b,
          n3wa, n3wb, n3wc, n3wd, n3b,
          n4w, n4b,
          o_ref):
    rng = range(_BG)
    xms = []
    for g in rng:
        x = x_ref[g]
        h = _lrelu(_bdot(x, c1w1[...]) + c1b1[...])
        h = _lrelu(_bdot(h, c1w2[...]) + c1b2[...])
        h = _lrelu(h)
        h = _lrelu(_bdot(h, c2w1[...]) + c2b1[...])
        h = _lrelu(_bdot(h, c2w2[...]) + c2b2[...])
        xms.append(x * (h > 0).astype(_f32))

    feats = xms
    hist = [xms]
    for w1, b1, w2, b2 in ((v1w1, v1b1, v1w2, v1b2), (v2w1, v2b1, v2w2, v2b2),
                           (v3w1, v3b1, v3w2, v3b2), (v4w1, v4b1, v4w2, v4b2)):
        sel4s = [_topk_sels(feats[g][:, 0:3]) for g in rng]
        feats = [_edge_conv(feats[g], sel4s[g], w1[...], b1[...], w2[...], b2[...])
                 for g in rng]
        hist.append(feats)

    outs = []
    for g in rng:
        xm, a, b, c, d = (hist[0][g], hist[1][g], hist[2][g], hist[3][g],
                          hist[4][g])
        h = (_bdot(xm, n1wx[...]) + _bdot(a, n1wa[...]) + _bdot(b, n1wb[...])
             + _bdot(c, n1wc[...]) + _bdot(d, n1wd[...]) + n1b[...])
        h = _lrelu(h)
        h = _bdot(h, n2w[...]) + n2b[...]

        ga = jnp.max(h, axis=0, keepdims=True)
        gb = jnp.min(h, axis=0, keepdims=True)
        gc = jnp.sum(h, axis=0, keepdims=True)
        gd = gc / _f32(_NPG)
        t = (_bdot(_lrelu(ga), n3wa[...]) + _bdot(_lrelu(gb), n3wb[...])
             + _bdot(_lrelu(gc), n3wc[...]) + _bdot(_lrelu(gd), n3wd[...])
             + n3b[...])
        t = _lrelu(t)
        out = _bdot(t, n4w[...]) + n4b[...]
        lane = lax.broadcasted_iota(jnp.int32, (1, 3), 1)
        outs.append(jnp.where(lane < 2, jnp.tanh(out), out))
    for g in rng:
        o_ref[g] = outs[g]


def kernel(x, edge_index, batch, params):
    del edge_index, batch
    p = params

    def full(arr):
        return pl.BlockSpec(arr.shape, lambda *_: (0,) * arr.ndim)

    def b2(name):
        return p[name].reshape(1, -1)

    def wb(arr):
        return arr.astype(_bf16)

    weights = []
    for pre in ('clean1', 'clean2'):
        weights += [wb(p[pre + '_W1']), b2(pre + '_b1'),
                    wb(p[pre + '_W2']), b2(pre + '_b2')]
    for pre in ('conv1', 'conv2', 'conv3', 'conv4'):
        weights += [wb(p[pre + '_W1']), b2(pre + '_b1'),
                    wb(p[pre + '_W2']), b2(pre + '_b2')]
    w = p['nn1_W']
    weights += [wb(w[0:5]), wb(w[5:197]), wb(w[197:389]), wb(w[389:581]),
                wb(w[581:773]), b2('nn1_b')]
    weights += [wb(p['nn2_W']), b2('nn2_b')]
    w = p['nn3_W']
    weights += [wb(w[0:192]), wb(w[192:384]), wb(w[384:576]), wb(w[576:768]),
                b2('nn3_b')]
    weights += [wb(p['nn4_W']), b2('nn4_b')]

    x3 = x.reshape(_NGRAPH, _NPG, 5)
    out = pl.pallas_call(
        _body,
        grid=(_NGRAPH // _BG,),
        in_specs=[pl.BlockSpec((_BG, _NPG, 5), lambda i: (i, 0, 0))]
                 + [full(wt) for wt in weights],
        out_specs=pl.BlockSpec((_BG, 1, 3), lambda i: (i, 0, 0)),
        out_shape=jax.ShapeDtypeStruct((_NGRAPH, 1, 3), _f32),
    )(x3, *weights)
    return out.reshape(_NGRAPH, 3)


# speedup vs baseline: 1.7333x; 1.0060x over previous
import jax
import jax.numpy as jnp
from jax import lax
from jax.experimental import pallas as pl

_N = 10000
_NGRAPH = 100
_NPG = 100
_K = 4
_BG = 4

_f32 = jnp.float32
_bf16 = jnp.bfloat16
_EXACT = lax.Precision.HIGHEST


def _lrelu(h):
    return jnp.maximum(h, 0.01 * h)


def _bdot(a, w_bf16):
    return jnp.dot(a.astype(_bf16), w_bf16, preferred_element_type=_f32)


def _topk_sels(pos):
    n = _NPG
    pos_t = jnp.transpose(pos)
    d2 = None
    for c in range(3):
        dc = pos[:, c:c + 1] - pos_t[c:c + 1, :]
        dc = dc * dc
        d2 = dc if d2 is None else d2 + dc
    ii = lax.broadcasted_iota(jnp.int32, (n, n), 0)
    jj = lax.broadcasted_iota(jnp.int32, (n, n), 1)
    d2 = d2 + jnp.where(ii == jj, _f32(1e10), _f32(0.0))
    idxs = []
    for _ in range(_K):
        m = jnp.min(d2, axis=1, keepdims=True)
        cand = jnp.where(d2 <= m, jj, n)
        idx = jnp.min(cand, axis=1, keepdims=True)
        idxs.append(idx)
        d2 = jnp.where(jj == idx, _f32(jnp.inf), d2)
    idx_cat = jnp.concatenate(idxs, axis=0)
    jj4 = lax.broadcasted_iota(jnp.int32, (_K * n, n), 1)
    return (jj4 == idx_cat).astype(_bf16)


def _edge_conv(feat, sel4, w1, b1, w2, b2):
    n = _NPG
    hi = feat.astype(_bf16)
    r1 = feat - hi.astype(_f32)
    md = r1.astype(_bf16)
    lo = (r1 - md.astype(_f32)).astype(_bf16)
    xj = (jnp.dot(sel4, hi, preferred_element_type=_f32)
          + jnp.dot(sel4, md, preferred_element_type=_f32)
          + jnp.dot(sel4, lo, preferred_element_type=_f32))
    xi = jnp.concatenate([feat] * _K, axis=0)
    e = jnp.concatenate([jnp.concatenate([hi] * _K, axis=0),
                         (xj - xi).astype(_bf16)], axis=1)
    h = _lrelu(jnp.dot(e, w1, preferred_element_type=_f32) + b1)
    h = _lrelu(_bdot(h, w2) + b2)
    return ((h[0 * n:1 * n] + h[1 * n:2 * n])
            + h[2 * n:3 * n]) + h[3 * n:4 * n]


def _body(x_ref,
          c1w1, c1b1, c1w2, c1b2,
          c2w1, c2b1, c2w2, c2b2,
          v1w1, v1b1, v1w2, v1b2,
          v2w1, v2b1, v2w2, v2b2,
          v3w1, v3b1, v3w2, v3b2,
          v4w1, v4b1, v4w2, v4b2,
          n1wx, n1wa, n1wb, n1wc, n1wd, n1b,
          n2w, n2b,
          n3wa, n3wb, n3wc, n3wd, n3b,
          n4w, n4b,
          o_ref):
    rng = range(_BG)
    xms = []
    for g in rng:
        x = x_ref[g]
        h = _lrelu(_bdot(x, c1w1[...]) + c1b1[...])
        h = _lrelu(_bdot(h, c1w2[...]) + c1b2[...])
        h = _lrelu(h)
        h = _lrelu(_bdot(h, c2w1[...]) + c2b1[...])
        h = _lrelu(_bdot(h, c2w2[...]) + c2b2[...])
        xms.append(x * (h > 0).astype(_f32))

    feats = xms
    hist = [xms]
    for w1, b1, w2, b2 in ((v1w1, v1b1, v1w2, v1b2), (v2w1, v2b1, v2w2, v2b2),
                           (v3w1, v3b1, v3w2, v3b2), (v4w1, v4b1, v4w2, v4b2)):
        sel4s = [_topk_sels(feats[g][:, 0:3]) for g in rng]
        feats = [_edge_conv(feats[g], sel4s[g], w1[...], b1[...], w2[...], b2[...])
                 for g in rng]
        hist.append(feats)

    outs = []
    for g in rng:
        xm, a, b, c, d = (hist[0][g], hist[1][g], hist[2][g], hist[3][g],
                          hist[4][g])
        h = (_bdot(xm, n1wx[...]) + _bdot(a, n1wa[...]) + _bdot(b, n1wb[...])
             + _bdot(c, n1wc[...]) + _bdot(d, n1wd[...]) + n1b[...])
        h = _lrelu(h)
        h = _bdot(h, n2w[...]) + n2b[...]

        ga = jnp.max(h, axis=0, keepdims=True)
        gb = jnp.min(h, axis=0, keepdims=True)
        gc = jnp.sum(h, axis=0, keepdims=True)
        gd = gc / _f32(_NPG)
        t = (_bdot(_lrelu(ga), n3wa[...]) + _bdot(_lrelu(gb), n3wb[...])
             + _bdot(_lrelu(gc), n3wc[...]) + _bdot(_lrelu(gd), n3wd[...])
             + n3b[...])
        t = _lrelu(t)
        out = _bdot(t, n4w[...]) + n4b[...]
        lane = lax.broadcasted_iota(jnp.int32, (1, 3), 1)
        outs.append(jnp.where(lane < 2, jnp.tanh(out), out))
    for g in rng:
        o_ref[g] = outs[g]


def kernel(x, edge_index, batch, params):
    del edge_index, batch
    p = params

    def full(arr):
        return pl.BlockSpec(arr.shape, lambda *_: (0,) * arr.ndim)

    def b2(name):
        return p[name].reshape(1, -1)

    def wb(arr):
        return arr.astype(_bf16)

    weights = []
    for pre in ('clean1', 'clean2'):
        weights += [wb(p[pre + '_W1']), b2(pre + '_b1'),
                    wb(p[pre + '_W2']), b2(pre + '_b2')]
    for pre in ('conv1', 'conv2', 'conv3', 'conv4'):
        weights += [wb(p[pre + '_W1']), b2(pre + '_b1'),
                    wb(p[pre + '_W2']), b2(pre + '_b2')]
    w = p['nn1_W']
    weights += [wb(w[0:5]), wb(w[5:197]), wb(w[197:389]), wb(w[389:581]),
                wb(w[581:773]), b2('nn1_b')]
    weights += [wb(p['nn2_W']), b2('nn2_b')]
    w = p['nn3_W']
    weights += [wb(w[0:192]), wb(w[192:384]), wb(w[384:576]), wb(w[576:768]),
                b2('nn3_b')]
    weights += [wb(p['nn4_W']), b2('nn4_b')]

    x3 = x.reshape(_NGRAPH, _NPG, 5)
    out = pl.pallas_call(
        _body,
        grid=(_NGRAPH // _BG,),
        in_specs=[pl.BlockSpec((_BG, _NPG, 5), lambda i: (i, 0, 0))]
                 + [full(wt) for wt in weights],
        out_specs=pl.BlockSpec((_BG, 1, 3), lambda i: (i, 0, 0)),
        out_shape=jax.ShapeDtypeStruct((_NGRAPH, 1, 3), _f32),
    )(x3, *weights)
    return out.reshape(_NGRAPH, 3)


# speedup vs baseline: 2.4895x; 1.4363x over previous
import jax
import jax.numpy as jnp
from jax import lax
from jax.experimental import pallas as pl

_N = 10000
_NGRAPH = 100
_NPG = 100
_K = 4
_BG = 4

_f32 = jnp.float32
_bf16 = jnp.bfloat16
_EXACT = lax.Precision.HIGHEST


def _lrelu(h):
    return jnp.maximum(h, 0.01 * h)


def _bdot(a, w_bf16):
    return jnp.dot(a.astype(_bf16), w_bf16, preferred_element_type=_f32)


def _topk_sels(pos):
    n = _NPG
    pos_t = jnp.transpose(pos)
    d2 = None
    for c in range(3):
        dc = pos[:, c:c + 1] - pos_t[c:c + 1, :]
        dc = dc * dc
        d2 = dc if d2 is None else d2 + dc
    ii = lax.broadcasted_iota(jnp.int32, (n, n), 0)
    jj = lax.broadcasted_iota(jnp.int32, (n, n), 1)
    d2 = d2 + jnp.where(ii == jj, _f32(1e10), _f32(0.0))
    idxs = []
    for _ in range(_K):
        m = jnp.min(d2, axis=0, keepdims=True)
        cand = jnp.where(d2 <= m, ii, n)
        idx = jnp.min(cand, axis=0, keepdims=True)
        idxs.append(jnp.transpose(idx))
        d2 = jnp.where(ii == idx, _f32(jnp.inf), d2)
    idx_cat = jnp.concatenate(idxs, axis=0)
    jj4 = lax.broadcasted_iota(jnp.int32, (_K * n, n), 1)
    return (jj4 == idx_cat).astype(_bf16)


def _edge_conv(feat, sel4, w1, b1, w2, b2):
    n = _NPG
    hi = feat.astype(_bf16)
    r1 = feat - hi.astype(_f32)
    md = r1.astype(_bf16)
    lo = (r1 - md.astype(_f32)).astype(_bf16)
    xj = (jnp.dot(sel4, hi, preferred_element_type=_f32)
          + jnp.dot(sel4, md, preferred_element_type=_f32)
          + jnp.dot(sel4, lo, preferred_element_type=_f32))
    xi = jnp.concatenate([feat] * _K, axis=0)
    e = jnp.concatenate([jnp.concatenate([hi] * _K, axis=0),
                         (xj - xi).astype(_bf16)], axis=1)
    h = _lrelu(jnp.dot(e, w1, preferred_element_type=_f32) + b1)
    h = _lrelu(_bdot(h, w2) + b2)
    return ((h[0 * n:1 * n] + h[1 * n:2 * n])
            + h[2 * n:3 * n]) + h[3 * n:4 * n]


def _body(x_ref,
          c1w1, c1b1, c1w2, c1b2,
          c2w1, c2b1, c2w2, c2b2,
          v1w1, v1b1, v1w2, v1b2,
          v2w1, v2b1, v2w2, v2b2,
          v3w1, v3b1, v3w2, v3b2,
          v4w1, v4b1, v4w2, v4b2,
          n1wx, n1wa, n1wb, n1wc, n1wd, n1b,
          n2w, n2b,
          n3wa, n3wb, n3wc, n3wd, n3b,
          n4w, n4b,
          o_ref):
    rng = range(_BG)
    xms = []
    for g in rng:
        x = x_ref[g]
        h = _lrelu(_bdot(x, c1w1[...]) + c1b1[...])
        h = _lrelu(_bdot(h, c1w2[...]) + c1b2[...])
        h = _lrelu(h)
        h = _lrelu(_bdot(h, c2w1[...]) + c2b1[...])
        h = _lrelu(_bdot(h, c2w2[...]) + c2b2[...])
        xms.append(x * (h > 0).astype(_f32))

    feats = xms
    hist = [xms]
    for w1, b1, w2, b2 in ((v1w1, v1b1, v1w2, v1b2), (v2w1, v2b1, v2w2, v2b2),
                           (v3w1, v3b1, v3w2, v3b2), (v4w1, v4b1, v4w2, v4b2)):
        sel4s = [_topk_sels(feats[g][:, 0:3]) for g in rng]
        feats = [_edge_conv(feats[g], sel4s[g], w1[...], b1[...], w2[...], b2[...])
                 for g in rng]
        hist.append(feats)

    outs = []
    for g in rng:
        xm, a, b, c, d = (hist[0][g], hist[1][g], hist[2][g], hist[3][g],
                          hist[4][g])
        h = (_bdot(xm, n1wx[...]) + _bdot(a, n1wa[...]) + _bdot(b, n1wb[...])
             + _bdot(c, n1wc[...]) + _bdot(d, n1wd[...]) + n1b[...])
        h = _lrelu(h)
        h = _bdot(h, n2w[...]) + n2b[...]

        ga = jnp.max(h, axis=0, keepdims=True)
        gb = jnp.min(h, axis=0, keepdims=True)
        gc = jnp.sum(h, axis=0, keepdims=True)
        gd = gc / _f32(_NPG)
        t = (_bdot(_lrelu(ga), n3wa[...]) + _bdot(_lrelu(gb), n3wb[...])
             + _bdot(_lrelu(gc), n3wc[...]) + _bdot(_lrelu(gd), n3wd[...])
             + n3b[...])
        t = _lrelu(t)
        out = _bdot(t, n4w[...]) + n4b[...]
        lane = lax.broadcasted_iota(jnp.int32, (1, 3), 1)
        outs.append(jnp.where(lane < 2, jnp.tanh(out), out))
    for g in rng:
        o_ref[g] = outs[g]


def kernel(x, edge_index, batch, params):
    del edge_index, batch
    p = params

    def full(arr):
        return pl.BlockSpec(arr.shape, lambda *_: (0,) * arr.ndim)

    def b2(name):
        return p[name].reshape(1, -1)

    def wb(arr):
        return arr.astype(_bf16)

    weights = []
    for pre in ('clean1', 'clean2'):
        weights += [wb(p[pre + '_W1']), b2(pre + '_b1'),
                    wb(p[pre + '_W2']), b2(pre + '_b2')]
    for pre in ('conv1', 'conv2', 'conv3', 'conv4'):
        weights += [wb(p[pre + '_W1']), b2(pre + '_b1'),
                    wb(p[pre + '_W2']), b2(pre + '_b2')]
    w = p['nn1_W']
    weights += [wb(w[0:5]), wb(w[5:197]), wb(w[197:389]), wb(w[389:581]),
                wb(w[581:773]), b2('nn1_b')]
    weights += [wb(p['nn2_W']), b2('nn2_b')]
    w = p['nn3_W']
    weights += [wb(w[0:192]), wb(w[192:384]), wb(w[384:576]), wb(w[576:768]),
                b2('nn3_b')]
    weights += [wb(p['nn4_W']), b2('nn4_b')]

    x3 = x.reshape(_NGRAPH, _NPG, 5)
    out = pl.pallas_call(
        _body,
        grid=(_NGRAPH // _BG,),
        in_specs=[pl.BlockSpec((_BG, _NPG, 5), lambda i: (i, 0, 0))]
                 + [full(wt) for wt in weights],
        out_specs=pl.BlockSpec((_BG, 1, 3), lambda i: (i, 0, 0)),
        out_shape=jax.ShapeDtypeStruct((_NGRAPH, 1, 3), _f32),
    )(x3, *weights)
    return out.reshape(_NGRAPH, 3)


# speedup vs baseline: 2.8524x; 1.1458x over previous
import jax
import jax.numpy as jnp
from jax import lax
from jax.experimental import pallas as pl

_N = 10000
_NGRAPH = 100
_NPG = 100
_K = 4
_NP = 104
_BG = 4

_f32 = jnp.float32
_bf16 = jnp.bfloat16
_EXACT = lax.Precision.HIGHEST


def _lrelu(h):
    return jnp.maximum(h, 0.01 * h)


def _bdot(a, w_bf16):
    return jnp.dot(a.astype(_bf16), w_bf16, preferred_element_type=_f32)


def _topk_sels(pos):
    n = _NP
    pos_t = jnp.transpose(pos)
    d2 = None
    for c in range(3):
        dc = pos[:, c:c + 1] - pos_t[c:c + 1, :]
        dc = dc * dc
        d2 = dc if d2 is None else d2 + dc
    ii = lax.broadcasted_iota(jnp.int32, (n, n), 0)
    jj = lax.broadcasted_iota(jnp.int32, (n, n), 1)
    d2 = d2 + jnp.where(ii == jj, _f32(1e10), _f32(0.0))
    d2 = jnp.where(ii >= _NPG, _f32(jnp.inf), d2)
    idxs = []
    for _ in range(_K):
        m = jnp.min(d2, axis=0, keepdims=True)
        cand = jnp.where(d2 <= m, ii, n)
        idx = jnp.min(cand, axis=0, keepdims=True)
        idxs.append(jnp.transpose(idx))
        d2 = jnp.where(ii == idx, _f32(jnp.inf), d2)
    idx_cat = jnp.concatenate(idxs, axis=0)
    jj4 = lax.broadcasted_iota(jnp.int32, (_K * n, n), 1)
    return (jj4 == idx_cat).astype(_bf16)


def _edge_conv(feat, sel4, w1, b1, w2, b2):
    n = _NP
    hi = feat.astype(_bf16)
    r1 = feat - hi.astype(_f32)
    md = r1.astype(_bf16)
    lo = (r1 - md.astype(_f32)).astype(_bf16)
    xj = (jnp.dot(sel4, hi, preferred_element_type=_f32)
          + jnp.dot(sel4, md, preferred_element_type=_f32)
          + jnp.dot(sel4, lo, preferred_element_type=_f32))
    xi = jnp.concatenate([feat] * _K, axis=0)
    e = jnp.concatenate([jnp.concatenate([hi] * _K, axis=0),
                         (xj - xi).astype(_bf16)], axis=1)
    h = _lrelu(jnp.dot(e, w1, preferred_element_type=_f32) + b1)
    h = _lrelu(_bdot(h, w2) + b2)
    return ((h[0 * n:1 * n] + h[1 * n:2 * n])
            + h[2 * n:3 * n]) + h[3 * n:4 * n]


def _body(x_ref,
          c1w1, c1b1, c1w2, c1b2,
          c2w1, c2b1, c2w2, c2b2,
          v1w1, v1b1, v1w2, v1b2,
          v2w1, v2b1, v2w2, v2b2,
          v3w1, v3b1, v3w2, v3b2,
          v4w1, v4b1, v4w2, v4b2,
          n1wx, n1wa, n1wb, n1wc, n1wd, n1b,
          n2w, n2b,
          n3wa, n3wb, n3wc, n3wd, n3b,
          n4w, n4b,
          o_ref):
    rng = range(_BG)
    xms = []
    for g in rng:
        x = x_ref[g]
        h = _lrelu(_bdot(x, c1w1[...]) + c1b1[...])
        h = _lrelu(_bdot(h, c1w2[...]) + c1b2[...])
        h = _lrelu(h)
        h = _lrelu(_bdot(h, c2w1[...]) + c2b1[...])
        h = _lrelu(_bdot(h, c2w2[...]) + c2b2[...])
        xms.append(x * (h > 0).astype(_f32))

    feats = xms
    hist = [xms]
    for w1, b1, w2, b2 in ((v1w1, v1b1, v1w2, v1b2), (v2w1, v2b1, v2w2, v2b2),
                           (v3w1, v3b1, v3w2, v3b2), (v4w1, v4b1, v4w2, v4b2)):
        nxt = []
        for g in rng:
            sel4 = _topk_sels(feats[g][:, 0:3])
            nxt.append(_edge_conv(feats[g], sel4, w1[...], b1[...], w2[...], b2[...]))
        feats = nxt
        hist.append(feats)

    outs = []
    for g in rng:
        xm, a, b, c, d = (hist[0][g], hist[1][g], hist[2][g], hist[3][g],
                          hist[4][g])
        h = (_bdot(xm, n1wx[...]) + _bdot(a, n1wa[...]) + _bdot(b, n1wb[...])
             + _bdot(c, n1wc[...]) + _bdot(d, n1wd[...]) + n1b[...])
        h = _lrelu(h)
        h = _bdot(h, n2w[...]) + n2b[...]

        rm = lax.broadcasted_iota(jnp.int32, h.shape, 0) < _NPG
        ga = jnp.max(jnp.where(rm, h, -jnp.inf), axis=0, keepdims=True)
        gb = jnp.min(jnp.where(rm, h, jnp.inf), axis=0, keepdims=True)
        gc = jnp.sum(jnp.where(rm, h, _f32(0.0)), axis=0, keepdims=True)
        gd = gc / _f32(_NPG)
        t = (_bdot(_lrelu(ga), n3wa[...]) + _bdot(_lrelu(gb), n3wb[...])
             + _bdot(_lrelu(gc), n3wc[...]) + _bdot(_lrelu(gd), n3wd[...])
             + n3b[...])
        t = _lrelu(t)
        out = _bdot(t, n4w[...]) + n4b[...]
        lane = lax.broadcasted_iota(jnp.int32, (1, 3), 1)
        outs.append(jnp.where(lane < 2, jnp.tanh(out), out))
    for g in rng:
        o_ref[g] = outs[g]


def kernel(x, edge_index, batch, params):
    del edge_index, batch
    p = params

    def full(arr):
        return pl.BlockSpec(arr.shape, lambda *_: (0,) * arr.ndim)

    def b2(name):
        return p[name].reshape(1, -1)

    def wb(arr):
        return arr.astype(_bf16)

    weights = []
    for pre in ('clean1', 'clean2'):
        weights += [wb(p[pre + '_W1']), b2(pre + '_b1'),
                    wb(p[pre + '_W2']), b2(pre + '_b2')]
    for pre in ('conv1', 'conv2', 'conv3', 'conv4'):
        weights += [wb(p[pre + '_W1']), b2(pre + '_b1'),
                    wb(p[pre + '_W2']), b2(pre + '_b2')]
    w = p['nn1_W']
    weights += [wb(w[0:5]), wb(w[5:197]), wb(w[197:389]), wb(w[389:581]),
                wb(w[581:773]), b2('nn1_b')]
    weights += [wb(p['nn2_W']), b2('nn2_b')]
    w = p['nn3_W']
    weights += [wb(w[0:192]), wb(w[192:384]), wb(w[384:576]), wb(w[576:768]),
                b2('nn3_b')]
    weights += [wb(p['nn4_W']), b2('nn4_b')]

    x3 = jnp.pad(x.reshape(_NGRAPH, _NPG, 5),
                 ((0, 0), (0, _NP - _NPG), (0, 0)))
    out = pl.pallas_call(
        _body,
        grid=(_NGRAPH // _BG,),
        in_specs=[pl.BlockSpec((_BG, _NP, 5), lambda i: (i, 0, 0))]
                 + [full(wt) for wt in weights],
        out_specs=pl.BlockSpec((_BG, 1, 3), lambda i: (i, 0, 0)),
        out_shape=jax.ShapeDtypeStruct((_NGRAPH, 1, 3), _f32),
    )(x3, *weights)
    return out.reshape(_NGRAPH, 3)


# speedup vs baseline: 2.9198x; 1.0236x over previous
import jax
import jax.numpy as jnp
from jax import lax
from jax.experimental import pallas as pl

_N = 10000
_NGRAPH = 100
_NPG = 100
_K = 4
_NP = 104
_BG = 5

_f32 = jnp.float32
_bf16 = jnp.bfloat16
_EXACT = lax.Precision.HIGHEST


def _lrelu(h):
    return jnp.maximum(h, 0.01 * h)


def _bdot(a, w_bf16):
    return jnp.dot(a.astype(_bf16), w_bf16, preferred_element_type=_f32)


def _topk_sels(pos):
    n = _NP
    pos_t = jnp.transpose(pos)
    d2 = None
    for c in range(3):
        dc = pos[:, c:c + 1] - pos_t[c:c + 1, :]
        dc = dc * dc
        d2 = dc if d2 is None else d2 + dc
    ii = lax.broadcasted_iota(jnp.int32, (n, n), 0)
    jj = lax.broadcasted_iota(jnp.int32, (n, n), 1)
    d2 = d2 + jnp.where(ii == jj, _f32(1e10), _f32(0.0))
    d2 = jnp.where(ii >= _NPG, _f32(jnp.inf), d2)
    idxs = []
    for _ in range(_K):
        m = jnp.min(d2, axis=0, keepdims=True)
        cand = jnp.where(d2 <= m, ii, n)
        idx = jnp.min(cand, axis=0, keepdims=True)
        idxs.append(jnp.transpose(idx))
        d2 = jnp.where(ii == idx, _f32(jnp.inf), d2)
    idx_cat = jnp.concatenate(idxs, axis=0)
    jj4 = lax.broadcasted_iota(jnp.int32, (_K * n, n), 1)
    return (jj4 == idx_cat).astype(_bf16)


def _edge_conv(feat, sel4, w1, b1, w2, b2):
    n = _NP
    hi = feat.astype(_bf16)
    r1 = feat - hi.astype(_f32)
    md = r1.astype(_bf16)
    lo = (r1 - md.astype(_f32)).astype(_bf16)
    xj = (jnp.dot(sel4, hi, preferred_element_type=_f32)
          + jnp.dot(sel4, md, preferred_element_type=_f32)
          + jnp.dot(sel4, lo, preferred_element_type=_f32))
    xi = jnp.concatenate([feat] * _K, axis=0)
    e = jnp.concatenate([jnp.concatenate([hi] * _K, axis=0),
                         (xj - xi).astype(_bf16)], axis=1)
    h = _lrelu(jnp.dot(e, w1, preferred_element_type=_f32) + b1)
    h = _lrelu(_bdot(h, w2) + b2)
    return ((h[0 * n:1 * n] + h[1 * n:2 * n])
            + h[2 * n:3 * n]) + h[3 * n:4 * n]


def _body(x_ref,
          c1w1, c1b1, c1w2, c1b2,
          c2w1, c2b1, c2w2, c2b2,
          v1w1, v1b1, v1w2, v1b2,
          v2w1, v2b1, v2w2, v2b2,
          v3w1, v3b1, v3w2, v3b2,
          v4w1, v4b1, v4w2, v4b2,
          n1wx, n1wa, n1wb, n1wc, n1wd, n1b,
          n2w, n2b,
          n3wa, n3wb, n3wc, n3wd, n3b,
          n4w, n4b,
          o_ref):
    rng = range(_BG)
    xms = []
    for g in rng:
        x = x_ref[g]
        h = _lrelu(_bdot(x, c1w1[...]) + c1b1[...])
        h = _lrelu(_bdot(h, c1w2[...]) + c1b2[...])
        h = _lrelu(h)
        h = _lrelu(_bdot(h, c2w1[...]) + c2b1[...])
        h = _lrelu(_bdot(h, c2w2[...]) + c2b2[...])
        xms.append(x * (h > 0).astype(_f32))

    feats = xms
    hist = [xms]
    for w1, b1, w2, b2 in ((v1w1, v1b1, v1w2, v1b2), (v2w1, v2b1, v2w2, v2b2),
                           (v3w1, v3b1, v3w2, v3b2), (v4w1, v4b1, v4w2, v4b2)):
        nxt = []
        for g in rng:
            sel4 = _topk_sels(feats[g][:, 0:3])
            nxt.append(_edge_conv(feats[g], sel4, w1[...], b1[...], w2[...], b2[...]))
        feats = nxt
        hist.append(feats)

    outs = []
    for g in rng:
        xm, a, b, c, d = (hist[0][g], hist[1][g], hist[2][g], hist[3][g],
                          hist[4][g])
        h = (_bdot(xm, n1wx[...]) + _bdot(a, n1wa[...]) + _bdot(b, n1wb[...])
             + _bdot(c, n1wc[...]) + _bdot(d, n1wd[...]) + n1b[...])
        h = _lrelu(h)
        h = _bdot(h, n2w[...]) + n2b[...]

        rm = lax.broadcasted_iota(jnp.int32, h.shape, 0) < _NPG
        ga = jnp.max(jnp.where(rm, h, -jnp.inf), axis=0, keepdims=True)
        gb = jnp.min(jnp.where(rm, h, jnp.inf), axis=0, keepdims=True)
        gc = jnp.sum(jnp.where(rm, h, _f32(0.0)), axis=0, keepdims=True)
        gd = gc / _f32(_NPG)
        t = (_bdot(_lrelu(ga), n3wa[...]) + _bdot(_lrelu(gb), n3wb[...])
             + _bdot(_lrelu(gc), n3wc[...]) + _bdot(_lrelu(gd), n3wd[...])
             + n3b[...])
        t = _lrelu(t)
        out = _bdot(t, n4w[...]) + n4b[...]
        lane = lax.broadcasted_iota(jnp.int32, (1, 3), 1)
        outs.append(jnp.where(lane < 2, jnp.tanh(out), out))
    for g in rng:
        o_ref[g] = outs[g]


def kernel(x, edge_index, batch, params):
    del edge_index, batch
    p = params

    def full(arr):
        return pl.BlockSpec(arr.shape, lambda *_: (0,) * arr.ndim)

    def b2(name):
        return p[name].reshape(1, -1)

    def wb(arr):
        return arr.astype(_bf16)

    weights = []
    for pre in ('clean1', 'clean2'):
        weights += [wb(p[pre + '_W1']), b2(pre + '_b1'),
                    wb(p[pre + '_W2']), b2(pre + '_b2')]
    for pre in ('conv1', 'conv2', 'conv3', 'conv4'):
        weights += [wb(p[pre + '_W1']), b2(pre + '_b1'),
                    wb(p[pre + '_W2']), b2(pre + '_b2')]
    w = p['nn1_W']
    weights += [wb(w[0:5]), wb(w[5:197]), wb(w[197:389]), wb(w[389:581]),
                wb(w[581:773]), b2('nn1_b')]
    weights += [wb(p['nn2_W']), b2('nn2_b')]
    w = p['nn3_W']
    weights += [wb(w[0:192]), wb(w[192:384]), wb(w[384:576]), wb(w[576:768]),
                b2('nn3_b')]
    weights += [wb(p['nn4_W']), b2('nn4_b')]

    x3 = jnp.pad(x.reshape(_NGRAPH, _NPG, 5),
                 ((0, 0), (0, _NP - _NPG), (0, 0)))
    out = pl.pallas_call(
        _body,
        grid=(_NGRAPH // _BG,),
        in_specs=[pl.BlockSpec((_BG, _NP, 5), lambda i: (i, 0, 0))]
                 + [full(wt) for wt in weights],
        out_specs=pl.BlockSpec((_BG, 1, 3), lambda i: (i, 0, 0)),
        out_shape=jax.ShapeDtypeStruct((_NGRAPH, 1, 3), _f32),
    )(x3, *weights)
    return out.reshape(_NGRAPH, 3)
